# baseline (speedup 1.0000x reference)
import functools

import jax
import jax.numpy as jnp
from jax import lax
from jax.experimental import pallas as pl
from jax.experimental.pallas import tpu as pltpu
from jax.experimental.pallas import tpu_sc as plsc

NP = 10240
EP = 163840
NWORK = 32
EPT = EP // NWORK
C = 128
NCH = EPT // C
NT_PER_CORE = 16
ROWS_PT = NP // NT_PER_CORE
NR = 9
NTY = 3
D = 128
RD = 64
NI = 8
BN = 1024


def _selu(x):
    alpha = 1.6732632423543772848170429916717
    scale = 1.0507009873554804934193349852946
    return scale * jnp.where(x > 0, x, alpha * (jnp.exp(x) - 1.0))


def _softmax(x):
    m = jnp.max(x, axis=-1, keepdims=True)
    e = jnp.exp(x - m)
    return e / jnp.sum(e, axis=-1, keepdims=True)


def _full_spec(shape):
    return pl.BlockSpec(shape, lambda b: tuple(0 for _ in shape))


def _dense_siw_body(p0, p1, tbrw, tbrb, rele, imw, imb, ibnw, ibnb,
                    s_out, iw_out):
    emb = p0[...] + p1[...]
    row16 = lax.broadcasted_iota(jnp.int32, (16, BN), 0)
    s_acc = jnp.zeros((16, BN), jnp.float32)
    for r in range(NR):
        h = _selu(jnp.dot(emb, tbrw[r], preferred_element_type=jnp.float32)
                  + tbrb[r][None, :])
        s_r = jnp.sum(h * rele[r][None, :], axis=-1)
        s_acc = s_acc + jnp.where(row16 == r, s_r[None, :], 0.0)
    s_out[...] = s_acc
    iw_rows = []
    for t in range(NTY):
        intents = _softmax(jnp.dot(emb, imw[t], preferred_element_type=jnp.float32)
                           + imb[t][None, :])
        w = _softmax(jnp.dot(intents, ibnw[t], preferred_element_type=jnp.float32)
                     + ibnb[t][None, :])
        iw_rows.append(jnp.concatenate(
            [w, jnp.zeros((BN, 16 - NR), jnp.float32)], axis=1))
    iw_out[...] = jnp.stack(iw_rows, axis=0)


def _dense_siw(p0, p1, tbrw, tbrb, rele, imw, imb, ibnw, ibnb):
    return pl.pallas_call(
        _dense_siw_body,
        grid=(NP // BN,),
        in_specs=[
            pl.BlockSpec((BN, D), lambda b: (b, 0)),
            pl.BlockSpec((BN, D), lambda b: (b, 0)),
            _full_spec((NR, D, RD)), _full_spec((NR, RD)), _full_spec((NR, RD)),
            _full_spec((NTY, D, NI)), _full_spec((NTY, NI)),
            _full_spec((NTY, NI, NR)), _full_spec((NTY, NR)),
        ],
        out_specs=[
            pl.BlockSpec((16, BN), lambda b: (0, b)),
            pl.BlockSpec((NTY, BN, 16), lambda b: (0, b, 0)),
        ],
        out_shape=[
            jax.ShapeDtypeStruct((16, NP), jnp.float32),
            jax.ShapeDtypeStruct((NTY, NP, 16), jnp.float32),
        ],
    )(p0, p1, tbrw, tbrb, rele, imw, imb, ibnw, ibnb)


def _dense_m_body(p0, p1, mfrw, m_out):
    emb = p0[...] + p1[...]
    m_rows = [jnp.dot(emb, mfrw[r], preferred_element_type=jnp.float32)
              for r in range(NR)]
    m_out[...] = jnp.stack(m_rows, axis=0)


def _dense_m(p0, p1, mfrw):
    return pl.pallas_call(
        _dense_m_body,
        grid=(NP // BN,),
        in_specs=[
            pl.BlockSpec((BN, D), lambda b: (b, 0)),
            pl.BlockSpec((BN, D), lambda b: (b, 0)),
            _full_spec((NR, D, D)),
        ],
        out_specs=pl.BlockSpec((NR, BN, D), lambda b: (0, b, 0)),
        out_shape=jax.ShapeDtypeStruct((NR, NP, D), jnp.float32),
    )(p0, p1, mfrw)


def _combine_body(p0, p1, out):
    out[...] = p0[...] + p1[...]


def _combine(p0, p1):
    return pl.pallas_call(
        _combine_body,
        grid=(NP // BN,),
        in_specs=[pl.BlockSpec((BN, D), lambda b: (b, 0))] * 2,
        out_specs=pl.BlockSpec((BN, D), lambda b: (b, 0)),
        out_shape=jax.ShapeDtypeStruct((NP, D), jnp.float32),
    )(p0, p1)


def _recip_body(d_ref, out_ref):
    d = d_ref[0:8, :] + d_ref[8:16, :]
    out_ref[...] = jnp.where(d == 0.0, 0.0, 1.0 / d)


def _recip(den):
    return pl.pallas_call(
        _recip_body,
        out_shape=jax.ShapeDtypeStruct((8, NP // 8), jnp.float32),
    )(den.reshape(16, NP // 8)).reshape(NP)


_MESH = plsc.VectorSubcoreMesh(core_axis_name="c", subcore_axis_name="s")


@functools.partial(
    pl.kernel,
    mesh=_MESH,
    out_type=[
        jax.ShapeDtypeStruct((EP,), jnp.float32),
        jax.ShapeDtypeStruct((2 * NP,), jnp.float32),
        jax.ShapeDtypeStruct((EP,), jnp.int32),
    ],
    scratch_types=[
        pltpu.VMEM((EPT,), jnp.int32),
        pltpu.VMEM((EPT,), jnp.int32),
        pltpu.VMEM((EPT,), jnp.int32),
        pltpu.VMEM((EPT,), jnp.int32),
        pltpu.VMEM((EPT,), jnp.int32),
        pltpu.VMEM((EPT,), jnp.float32),
        pltpu.VMEM((C,), jnp.int32),
        pltpu.VMEM((C,), jnp.int32),
        pltpu.VMEM((C,), jnp.int32),
        pltpu.VMEM((C,), jnp.int32),
        pltpu.VMEM((C,), jnp.int32),
        pltpu.VMEM((C,), jnp.int32),
        pltpu.VMEM((C,), jnp.float32),
        pltpu.VMEM((C,), jnp.float32),
        pltpu.VMEM((C,), jnp.float32),
        pltpu.VMEM((C,), jnp.float32),
        pltpu.VMEM((ROWS_PT,), jnp.float32),
        pltpu.VMEM_SHARED((NP,), jnp.float32),
        pltpu.SemaphoreType.DMA,
        pltpu.SemaphoreType.DMA,
    ],
)
def _sc_scores(src_h, tgt_h, tt_h, rel_h, sflat_h, iwflat_h,
               exp_h, den_h, sidx_h, srcall, tgtall, ttall, relall,
               sidxall, expall,
               sidx0, sidx1, iwiv0, iwiv1, tgtc0, tgtc1,
               sbuf0, sbuf1, iwbuf0, iwbuf1, zb, den_sh, sem0, sem1):
    cid = lax.axis_index("c")
    sid = lax.axis_index("s")
    wid = cid * NT_PER_CORE + sid
    base = wid * EPT
    slots = [(sidx0, iwiv0, tgtc0, sbuf0, iwbuf0, sem0),
             (sidx1, iwiv1, tgtc1, sbuf1, iwbuf1, sem1)]

    pltpu.sync_copy(src_h.at[pl.ds(base, EPT)], srcall)
    pltpu.sync_copy(tgt_h.at[pl.ds(base, EPT)], tgtall)
    pltpu.sync_copy(tt_h.at[pl.ds(base, EPT)], ttall)
    pltpu.sync_copy(rel_h.at[pl.ds(base, EPT)], relall)

    def zb_init(i, _):
        zb[pl.ds(i * 16, 16)] = jnp.zeros((16,), jnp.float32)
        return 0

    lax.fori_loop(0, ROWS_PT // 16, zb_init, 0)
    pltpu.sync_copy(zb, den_sh.at[pl.ds(sid * ROWS_PT, ROWS_PT)])
    plsc.subcore_barrier()

    def issue(c, b):
        sidx, iwiv, tgtc, sbuf, iwbuf, sem = slots[b]
        off = c * C
        for g in range(C // 16):
            sl = pl.ds(g * 16, 16)
            al = pl.ds(off + g * 16, 16)
            t16 = tgtall[al]
            r16 = relall[al]
            tgtc[sl] = t16
            iwiv[sl] = (ttall[al] * NP + t16) * 16 + r16
            mi = r16 * NP + srcall[al]
            sidx[sl] = mi
            sidxall[al] = mi
        pltpu.async_copy(sflat_h.at[sidx], sbuf, sem)
        pltpu.async_copy(iwflat_h.at[iwiv], iwbuf, sem)

    def wait(b):
        sidx, iwiv, tgtc, sbuf, iwbuf, sem = slots[b]
        pltpu.make_async_copy(sflat_h.at[sidx], sbuf, sem).wait()
        pltpu.make_async_copy(iwflat_h.at[iwiv], iwbuf, sem).wait()

    def process(c, b):
        sidx, iwiv, tgtc, sbuf, iwbuf, sem = slots[b]
        off = c * C
        for g in range(C // 16):
            sl = pl.ds(g * 16, 16)
            expall[pl.ds(off + g * 16, 16)] = jnp.exp(sbuf[sl] * iwbuf[sl])
        pltpu.sync_copy(expall.at[pl.ds(off, C)], den_sh.at[tgtc], add=True)

    issue(0, 0)

    def super_step(k, _):
        issue(2 * k + 1, 1)
        wait(0)
        process(2 * k, 0)

        @pl.when(k < NCH // 2 - 1)
        def _():
            issue(2 * k + 2, 0)

        wait(1)
        process(2 * k + 1, 1)
        return 0

    lax.fori_loop(0, NCH // 2, super_step, 0)
    pltpu.sync_copy(expall, exp_h.at[pl.ds(base, EPT)])
    pltpu.sync_copy(sidxall, sidx_h.at[pl.ds(base, EPT)])
    plsc.subcore_barrier()
    pltpu.sync_copy(den_sh.at[pl.ds(sid * ROWS_PT, ROWS_PT)],
                    den_h.at[pl.ds(cid * NP + sid * ROWS_PT, ROWS_PT)])


@functools.partial(
    pl.kernel,
    mesh=_MESH,
    out_type=[
        jax.ShapeDtypeStruct((EP,), jnp.float32),
        jax.ShapeDtypeStruct((2, NP, D), jnp.float32),
    ],
    scratch_types=[
        pltpu.VMEM((EPT,), jnp.int32),
        pltpu.VMEM((EPT,), jnp.int32),
        pltpu.VMEM((EPT,), jnp.float32),
        pltpu.VMEM((C,), jnp.int32),
        pltpu.VMEM((C,), jnp.int32),
        pltpu.VMEM((C,), jnp.int32),
        pltpu.VMEM((C,), jnp.int32),
        pltpu.VMEM((C,), jnp.float32),
        pltpu.VMEM((C,), jnp.float32),
        pltpu.VMEM((C, D), jnp.float32),
        pltpu.VMEM((C, D), jnp.float32),
        pltpu.VMEM_SHARED((NP, D), jnp.float32),
        pltpu.SemaphoreType.DMA,
        pltpu.SemaphoreType.DMA,
    ],
)
def _sc_messages(midx_h, tgt_h, exp_h, recip_h, m_h,
                 ew_h, out_h, midxall, tgtall, expall,
                 midx0, midx1, tgtc0, tgtc1,
                 rb0, rb1, mrows0, mrows1, out_sh, sem0, sem1):
    cid = lax.axis_index("c")
    sid = lax.axis_index("s")
    wid = cid * NT_PER_CORE + sid
    base = wid * EPT
    slots = [(midx0, tgtc0, rb0, mrows0, sem0),
             (midx1, tgtc1, rb1, mrows1, sem1)]

    pltpu.sync_copy(midx_h.at[pl.ds(base, EPT)], midxall)
    pltpu.sync_copy(tgt_h.at[pl.ds(base, EPT)], tgtall)
    pltpu.sync_copy(exp_h.at[pl.ds(base, EPT)], expall)

    def mz(i, _):
        for j in range(D // 16):
            mrows0[i, pl.ds(j * 16, 16)] = jnp.zeros((16,), jnp.float32)
        return 0

    lax.fori_loop(0, C, mz, 0)
    for k in range(ROWS_PT // C):
        pltpu.sync_copy(mrows0, out_sh.at[pl.ds(sid * ROWS_PT + k * C, C)])
    plsc.subcore_barrier()

    def issue(c, b):
        midx, tgtc, rb, mrows, sem = slots[b]
        off = c * C
        for g in range(C // 16):
            sl = pl.ds(g * 16, 16)
            al = pl.ds(off + g * 16, 16)
            tgtc[sl] = tgtall[al]
            midx[sl] = midxall[al]
        pltpu.async_copy(m_h.at[midx], mrows, sem)
        pltpu.async_copy(recip_h.at[tgtc], rb, sem)

    def wait(b):
        midx, tgtc, rb, mrows, sem = slots[b]
        pltpu.make_async_copy(m_h.at[midx], mrows, sem).wait()
        pltpu.make_async_copy(recip_h.at[tgtc], rb, sem).wait()

    def process(c, b):
        midx, tgtc, rb, mrows, sem = slots[b]
        off = c * C
        for g in range(C // 16):
            sl = pl.ds(g * 16, 16)
            al = pl.ds(off + g * 16, 16)
            expall[al] = expall[al] * rb[sl]

        def scale(g, _):
            w16 = expall[pl.ds(off + g * 16, 16)]
            for j in range(16):
                w = w16[j]
                e = g * 16 + j
                for k in range(D // 16):
                    sl = pl.ds(k * 16, 16)
                    mrows[e, sl] = mrows[e, sl] * w
            return 0

        lax.fori_loop(0, C // 16, scale, 0)
        pltpu.sync_copy(mrows, out_sh.at[tgtc], add=True)

    issue(0, 0)

    def super_step(k, _):
        issue(2 * k + 1, 1)
        wait(0)
        process(2 * k, 0)

        @pl.when(k < NCH // 2 - 1)
        def _():
            issue(2 * k + 2, 0)

        wait(1)
        process(2 * k + 1, 1)
        return 0

    lax.fori_loop(0, NCH // 2, super_step, 0)
    pltpu.sync_copy(expall, ew_h.at[pl.ds(base, EPT)])
    plsc.subcore_barrier()
    pltpu.sync_copy(out_sh.at[pl.ds(sid * ROWS_PT, ROWS_PT)],
                    out_h.at[cid, pl.ds(sid * ROWS_PT, ROWS_PT)])


def kernel(new_edges, new_node_types, entity_embeddings, default_emb,
           relation_emb_table, tbr_W, tbr_b, mfr_W, im_W, im_b, ibn_W, ibn_b):
    E = new_edges.shape[0] + 1
    static_edge = jnp.zeros((1, 5), jnp.int32)
    edges = jnp.concatenate([new_edges, static_edge], axis=0)
    npad = EP - E
    i = jnp.arange(npad, dtype=jnp.int32)
    src = jnp.concatenate([edges[:, 0], 10000 + (i % 16)])
    tgt = jnp.concatenate([edges[:, 2], 10016 + (i % 16)])
    tt = jnp.concatenate([edges[:, 3], jnp.zeros((npad,), jnp.int32)])
    rel = jnp.concatenate([edges[:, 4], jnp.zeros((npad,), jnp.int32)])

    emb = jnp.concatenate([default_emb[new_node_types], entity_embeddings], axis=0)
    p0 = jnp.pad(emb, ((0, NP - emb.shape[0]), (0, 0)))
    p1 = jnp.zeros((NP, D), jnp.float32)

    ews = []
    for _ in range(3):
        srows, iwp = _dense_siw(p0, p1, tbr_W, tbr_b, relation_emb_table,
                                im_W, im_b, ibn_W, ibn_b)
        m = _dense_m(p0, p1, mfr_W)
        sflat = srows.reshape(16 * NP)
        iwflat = iwp.reshape(NTY * NP * 16)
        mflat = m.reshape(NR * NP, D)
        exp_s, denoms, midx = _sc_scores(src, tgt, tt, rel, sflat, iwflat)
        recip = _recip(denoms)
        ew, outp = _sc_messages(midx, tgt, exp_s, recip, mflat)
        p0, p1 = outp[0], outp[1]
        ews.append(ew)

    ent = _combine(p0, p1)
    return ent[:10000], jnp.stack(ews)[:, :E]


# speedup vs baseline: 1.0320x; 1.0320x over previous
import functools

import jax
import jax.numpy as jnp
from jax import lax
from jax.experimental import pallas as pl
from jax.experimental.pallas import tpu as pltpu
from jax.experimental.pallas import tpu_sc as plsc

NP = 10240
EP = 163840
NWORK = 32
EPT = EP // NWORK
C = 128
NCH = EPT // C
NT_PER_CORE = 16
ROWS_PT = NP // NT_PER_CORE
NR = 9
NTY = 3
D = 128
RD = 64
NI = 8
BN = 1024


def _selu(x):
    alpha = 1.6732632423543772848170429916717
    scale = 1.0507009873554804934193349852946
    return scale * jnp.where(x > 0, x, alpha * (jnp.exp(x) - 1.0))


def _softmax(x):
    m = jnp.max(x, axis=-1, keepdims=True)
    e = jnp.exp(x - m)
    return e / jnp.sum(e, axis=-1, keepdims=True)


def _full_spec(shape):
    return pl.BlockSpec(shape, lambda b: tuple(0 for _ in shape))


def _dense_body(p0, p1, tbrw, tbrb, rele, mfrw, imw, imb, ibnw, ibnb,
                s_out, iw_out, m_out):
    emb = p0[...] + p1[...]
    row16 = lax.broadcasted_iota(jnp.int32, (16, BN), 0)
    s_acc = jnp.zeros((16, BN), jnp.float32)
    m_rows = []
    for r in range(NR):
        h = _selu(jnp.dot(emb, tbrw[r], preferred_element_type=jnp.float32)
                  + tbrb[r][None, :])
        s_r = jnp.sum(h * rele[r][None, :], axis=-1)
        s_acc = s_acc + jnp.where(row16 == r, s_r[None, :], 0.0)
        m_rows.append(jnp.dot(emb, mfrw[r], preferred_element_type=jnp.float32))
    s_out[...] = s_acc
    m_out[...] = jnp.stack(m_rows, axis=0)
    iw_rows = []
    for t in range(NTY):
        intents = _softmax(jnp.dot(emb, imw[t], preferred_element_type=jnp.float32)
                           + imb[t][None, :])
        w = _softmax(jnp.dot(intents, ibnw[t], preferred_element_type=jnp.float32)
                     + ibnb[t][None, :])
        iw_rows.append(jnp.concatenate(
            [w, jnp.zeros((BN, 16 - NR), jnp.float32)], axis=1))
    iw_out[...] = jnp.stack(iw_rows, axis=0)


def _dense(p0, p1, tbrw, tbrb, rele, mfrw, imw, imb, ibnw, ibnb):
    return pl.pallas_call(
        _dense_body,
        grid=(NP // BN,),
        in_specs=[
            pl.BlockSpec((BN, D), lambda b: (b, 0)),
            pl.BlockSpec((BN, D), lambda b: (b, 0)),
            _full_spec((NR, D, RD)), _full_spec((NR, RD)), _full_spec((NR, RD)),
            _full_spec((NR, D, D)), _full_spec((NTY, D, NI)),
            _full_spec((NTY, NI)), _full_spec((NTY, NI, NR)),
            _full_spec((NTY, NR)),
        ],
        out_specs=[
            pl.BlockSpec((16, BN), lambda b: (0, b)),
            pl.BlockSpec((NTY, BN, 16), lambda b: (0, b, 0)),
            pl.BlockSpec((NR, BN, D), lambda b: (0, b, 0)),
        ],
        out_shape=[
            jax.ShapeDtypeStruct((16, NP), jnp.float32),
            jax.ShapeDtypeStruct((NTY, NP, 16), jnp.float32),
            jax.ShapeDtypeStruct((NR, NP, D), jnp.float32),
        ],
    )(p0, p1, tbrw, tbrb, rele, mfrw, imw, imb, ibnw, ibnb)


def _combine_body(p0, p1, out):
    out[...] = p0[...] + p1[...]


def _combine(p0, p1):
    return pl.pallas_call(
        _combine_body,
        grid=(NP // BN,),
        in_specs=[pl.BlockSpec((BN, D), lambda b: (b, 0))] * 2,
        out_specs=pl.BlockSpec((BN, D), lambda b: (b, 0)),
        out_shape=jax.ShapeDtypeStruct((NP, D), jnp.float32),
    )(p0, p1)


_MESH = plsc.VectorSubcoreMesh(core_axis_name="c", subcore_axis_name="s")


@functools.partial(
    pl.kernel,
    mesh=_MESH,
    out_type=[
        jax.ShapeDtypeStruct((EP,), jnp.float32),
        jax.ShapeDtypeStruct((2 * NP,), jnp.float32),
        jax.ShapeDtypeStruct((EP,), jnp.int32),
    ],
    scratch_types=[
        pltpu.VMEM((EPT,), jnp.int32),
        pltpu.VMEM((NCH, C), jnp.int32),
        pltpu.VMEM((EPT,), jnp.int32),
        pltpu.VMEM((EPT,), jnp.int32),
        pltpu.VMEM((EPT,), jnp.int32),
        pltpu.VMEM((EPT,), jnp.float32),
        pltpu.VMEM((C,), jnp.int32),
        pltpu.VMEM((C,), jnp.int32),
        pltpu.VMEM((C,), jnp.int32),
        pltpu.VMEM((C,), jnp.int32),
        pltpu.VMEM((C,), jnp.float32),
        pltpu.VMEM((C,), jnp.float32),
        pltpu.VMEM((C,), jnp.float32),
        pltpu.VMEM((C,), jnp.float32),
        pltpu.VMEM((ROWS_PT,), jnp.float32),
        pltpu.VMEM_SHARED((NP,), jnp.float32),
        pltpu.SemaphoreType.DMA,
        pltpu.SemaphoreType.DMA,
    ],
)
def _sc_scores(src_h, tgt2_h, tt_h, rel_h, sflat_h, iwflat_h,
               exp_h, den_h, sidx_h, srcall, tgt2d, ttall, relall,
               sidxall, expall,
               sidx0, sidx1, iwiv0, iwiv1,
               sbuf0, sbuf1, iwbuf0, iwbuf1, zb, den_sh, sem0, sem1):
    cid = lax.axis_index("c")
    sid = lax.axis_index("s")
    wid = cid * NT_PER_CORE + sid
    base = wid * EPT
    slots = [(sidx0, iwiv0, sbuf0, iwbuf0, sem0),
             (sidx1, iwiv1, sbuf1, iwbuf1, sem1)]

    pltpu.sync_copy(src_h.at[pl.ds(base, EPT)], srcall)
    pltpu.sync_copy(tgt2_h.at[pl.ds(wid * NCH, NCH)], tgt2d)
    pltpu.sync_copy(tt_h.at[pl.ds(base, EPT)], ttall)
    pltpu.sync_copy(rel_h.at[pl.ds(base, EPT)], relall)

    def zb_init(i, _):
        zb[pl.ds(i * 16, 16)] = jnp.zeros((16,), jnp.float32)
        return 0

    lax.fori_loop(0, ROWS_PT // 16, zb_init, 0)
    pltpu.sync_copy(zb, den_sh.at[pl.ds(sid * ROWS_PT, ROWS_PT)])
    plsc.subcore_barrier()

    def issue(c, b):
        sidx, iwiv, sbuf, iwbuf, sem = slots[b]
        off = c * C
        for g in range(C // 16):
            sl = pl.ds(g * 16, 16)
            al = pl.ds(off + g * 16, 16)
            t16 = tgt2d[c, sl]
            r16 = relall[al]
            iwiv[sl] = (ttall[al] * NP + t16) * 16 + r16
            mi = r16 * NP + srcall[al]
            sidx[sl] = mi
            sidxall[al] = mi
        pltpu.async_copy(sflat_h.at[sidx], sbuf, sem)
        pltpu.async_copy(iwflat_h.at[iwiv], iwbuf, sem)

    def wait(b):
        sidx, iwiv, sbuf, iwbuf, sem = slots[b]
        pltpu.make_async_copy(sflat_h.at[sidx], sbuf, sem).wait()
        pltpu.make_async_copy(iwflat_h.at[iwiv], iwbuf, sem).wait()

    def process(c, b):
        sidx, iwiv, sbuf, iwbuf, sem = slots[b]
        off = c * C
        for g in range(C // 16):
            sl = pl.ds(g * 16, 16)
            expall[pl.ds(off + g * 16, 16)] = jnp.exp(sbuf[sl] * iwbuf[sl])
        pltpu.sync_copy(expall.at[pl.ds(off, C)],
                        den_sh.at[tgt2d.at[c]], add=True)

    issue(0, 0)

    def super_step(k, _):
        issue(2 * k + 1, 1)
        wait(0)
        process(2 * k, 0)

        @pl.when(k < NCH // 2 - 1)
        def _():
            issue(2 * k + 2, 0)

        wait(1)
        process(2 * k + 1, 1)
        return 0

    lax.fori_loop(0, NCH // 2, super_step, 0)
    pltpu.sync_copy(expall, exp_h.at[pl.ds(base, EPT)])
    pltpu.sync_copy(sidxall, sidx_h.at[pl.ds(base, EPT)])
    plsc.subcore_barrier()
    pltpu.sync_copy(den_sh.at[pl.ds(sid * ROWS_PT, ROWS_PT)],
                    den_h.at[pl.ds(cid * NP + sid * ROWS_PT, ROWS_PT)])


@functools.partial(
    pl.kernel,
    mesh=_MESH,
    out_type=[
        jax.ShapeDtypeStruct((EP,), jnp.float32),
        jax.ShapeDtypeStruct((2, NP, D), jnp.float32),
    ],
    scratch_types=[
        pltpu.VMEM((NCH, C), jnp.int32),
        pltpu.VMEM((NCH, C), jnp.int32),
        pltpu.VMEM((EPT,), jnp.float32),
        pltpu.VMEM((C,), jnp.int32),
        pltpu.VMEM((C,), jnp.int32),
        pltpu.VMEM((C,), jnp.float32),
        pltpu.VMEM((C,), jnp.float32),
        pltpu.VMEM((C,), jnp.float32),
        pltpu.VMEM((C,), jnp.float32),
        pltpu.VMEM((C, D), jnp.float32),
        pltpu.VMEM((C, D), jnp.float32),
        pltpu.VMEM_SHARED((NP, D), jnp.float32),
        pltpu.SemaphoreType.DMA,
        pltpu.SemaphoreType.DMA,
    ],
)
def _sc_messages(midx2_h, tgt2_h, exp_h, den_h, m_h,
                 ew_h, out_h, midx2d, tgt2d, expall,
                 tgt20, tgt21, db00, db01, db10, db11,
                 mrows0, mrows1, out_sh, sem0, sem1):
    cid = lax.axis_index("c")
    sid = lax.axis_index("s")
    wid = cid * NT_PER_CORE + sid
    base = wid * EPT
    slots = [(tgt20, db00, db10, mrows0, sem0),
             (tgt21, db01, db11, mrows1, sem1)]

    pltpu.sync_copy(midx2_h.at[pl.ds(wid * NCH, NCH)], midx2d)
    pltpu.sync_copy(tgt2_h.at[pl.ds(wid * NCH, NCH)], tgt2d)
    pltpu.sync_copy(exp_h.at[pl.ds(base, EPT)], expall)

    def mz(i, _):
        for j in range(D // 16):
            mrows0[i, pl.ds(j * 16, 16)] = jnp.zeros((16,), jnp.float32)
        return 0

    lax.fori_loop(0, C, mz, 0)
    for k in range(ROWS_PT // C):
        pltpu.sync_copy(mrows0, out_sh.at[pl.ds(sid * ROWS_PT + k * C, C)])
    plsc.subcore_barrier()

    def issue(c, b):
        tgt2, db0, db1, mrows, sem = slots[b]
        for g in range(C // 16):
            sl = pl.ds(g * 16, 16)
            tgt2[sl] = tgt2d[c, sl] + NP
        pltpu.async_copy(m_h.at[midx2d.at[c]], mrows, sem)
        pltpu.async_copy(den_h.at[tgt2d.at[c]], db0, sem)
        pltpu.async_copy(den_h.at[tgt2], db1, sem)

    def wait(b):
        tgt2, db0, db1, mrows, sem = slots[b]
        pltpu.make_async_copy(m_h.at[midx2d.at[0]], mrows, sem).wait()
        pltpu.make_async_copy(den_h.at[tgt2d.at[0]], db0, sem).wait()
        pltpu.make_async_copy(den_h.at[tgt2], db1, sem).wait()

    def process(c, b):
        tgt2, db0, db1, mrows, sem = slots[b]
        off = c * C
        for g in range(C // 16):
            sl = pl.ds(g * 16, 16)
            al = pl.ds(off + g * 16, 16)
            d = db0[sl] + db1[sl]
            expall[al] = expall[al] * jnp.where(d == 0.0, 0.0, 1.0 / d)

        def scale(g, _):
            w16 = expall[pl.ds(off + g * 16, 16)]
            for j in range(16):
                w = w16[j]
                e = g * 16 + j
                for k in range(D // 16):
                    sl = pl.ds(k * 16, 16)
                    mrows[e, sl] = mrows[e, sl] * w
            return 0

        lax.fori_loop(0, C // 16, scale, 0)
        pltpu.sync_copy(mrows, out_sh.at[tgt2d.at[c]], add=True)

    issue(0, 0)

    def super_step(k, _):
        issue(2 * k + 1, 1)
        wait(0)
        process(2 * k, 0)

        @pl.when(k < NCH // 2 - 1)
        def _():
            issue(2 * k + 2, 0)

        wait(1)
        process(2 * k + 1, 1)
        return 0

    lax.fori_loop(0, NCH // 2, super_step, 0)
    pltpu.sync_copy(expall, ew_h.at[pl.ds(base, EPT)])
    plsc.subcore_barrier()
    pltpu.sync_copy(out_sh.at[pl.ds(sid * ROWS_PT, ROWS_PT)],
                    out_h.at[cid, pl.ds(sid * ROWS_PT, ROWS_PT)])


def kernel(new_edges, new_node_types, entity_embeddings, default_emb,
           relation_emb_table, tbr_W, tbr_b, mfr_W, im_W, im_b, ibn_W, ibn_b):
    E = new_edges.shape[0] + 1
    static_edge = jnp.zeros((1, 5), jnp.int32)
    edges = jnp.concatenate([new_edges, static_edge], axis=0)
    npad = EP - E
    i = jnp.arange(npad, dtype=jnp.int32)
    src = jnp.concatenate([edges[:, 0], 10000 + (i % 16)])
    tgt = jnp.concatenate([edges[:, 2], 10016 + (i % 16)])
    tt = jnp.concatenate([edges[:, 3], jnp.zeros((npad,), jnp.int32)])
    rel = jnp.concatenate([edges[:, 4], jnp.zeros((npad,), jnp.int32)])

    emb = jnp.concatenate([default_emb[new_node_types], entity_embeddings], axis=0)
    p0 = jnp.pad(emb, ((0, NP - emb.shape[0]), (0, 0)))
    p1 = jnp.zeros((NP, D), jnp.float32)

    tgt2d = tgt.reshape(EP // C, C)
    ews = []
    for _ in range(3):
        srows, iwp, m = _dense(p0, p1, tbr_W, tbr_b, relation_emb_table,
                               mfr_W, im_W, im_b, ibn_W, ibn_b)
        sflat = srows.reshape(16 * NP)
        iwflat = iwp.reshape(NTY * NP * 16)
        mflat = m.reshape(NR * NP, D)
        exp_s, denoms, midx = _sc_scores(src, tgt2d, tt, rel, sflat, iwflat)
        ew, outp = _sc_messages(midx.reshape(EP // C, C), tgt2d, exp_s,
                                denoms, mflat)
        p0, p1 = outp[0], outp[1]
        ews.append(ew)

    ent = _combine(p0, p1)
    return ent[:10000], jnp.stack(ews)[:, :E]


# speedup vs baseline: 1.0500x; 1.0175x over previous
import functools

import jax
import jax.numpy as jnp
from jax import lax
from jax.experimental import pallas as pl
from jax.experimental.pallas import tpu as pltpu
from jax.experimental.pallas import tpu_sc as plsc

NP = 10240
EP = 163840
NWORK = 32
EPT = EP // NWORK
C = 128
NCH = EPT // C
NT_PER_CORE = 16
ROWS_PT = NP // NT_PER_CORE
NR = 9
NTY = 3
D = 128
RD = 64
NI = 8
BN = 1024


def _selu(x):
    alpha = 1.6732632423543772848170429916717
    scale = 1.0507009873554804934193349852946
    return scale * jnp.where(x > 0, x, alpha * (jnp.exp(x) - 1.0))


def _softmax(x):
    m = jnp.max(x, axis=-1, keepdims=True)
    e = jnp.exp(x - m)
    return e / jnp.sum(e, axis=-1, keepdims=True)


def _full_spec(shape):
    return pl.BlockSpec(shape, lambda b: tuple(0 for _ in shape))


def _dense_body(p0, p1, tbrw, tbrb, rele, mfrw, imw, imb, ibnw, ibnb,
                s_out, iw_out, m_out):
    emb = p0[...] + p1[...]
    row16 = lax.broadcasted_iota(jnp.int32, (16, BN), 0)
    s_acc = jnp.zeros((16, BN), jnp.float32)
    m_rows = []
    for r in range(NR):
        h = _selu(jnp.dot(emb, tbrw[r], preferred_element_type=jnp.float32)
                  + tbrb[r][None, :])
        s_r = jnp.sum(h * rele[r][None, :], axis=-1)
        s_acc = s_acc + jnp.where(row16 == r, s_r[None, :], 0.0)
        m_rows.append(jnp.dot(emb, mfrw[r], preferred_element_type=jnp.float32))
    s_out[...] = s_acc
    m_out[...] = jnp.stack(m_rows, axis=0)
    iw_rows = []
    for t in range(NTY):
        intents = _softmax(jnp.dot(emb, imw[t], preferred_element_type=jnp.float32)
                           + imb[t][None, :])
        w = _softmax(jnp.dot(intents, ibnw[t], preferred_element_type=jnp.float32)
                     + ibnb[t][None, :])
        iw_rows.append(jnp.concatenate(
            [w, jnp.zeros((BN, 16 - NR), jnp.float32)], axis=1))
    iw_out[...] = jnp.stack(iw_rows, axis=0)


def _dense(p0, p1, tbrw, tbrb, rele, mfrw, imw, imb, ibnw, ibnb):
    return pl.pallas_call(
        _dense_body,
        grid=(NP // BN,),
        in_specs=[
            pl.BlockSpec((BN, D), lambda b: (b, 0)),
            pl.BlockSpec((BN, D), lambda b: (b, 0)),
            _full_spec((NR, D, RD)), _full_spec((NR, RD)), _full_spec((NR, RD)),
            _full_spec((NR, D, D)), _full_spec((NTY, D, NI)),
            _full_spec((NTY, NI)), _full_spec((NTY, NI, NR)),
            _full_spec((NTY, NR)),
        ],
        out_specs=[
            pl.BlockSpec((16, BN), lambda b: (0, b)),
            pl.BlockSpec((NTY, BN, 16), lambda b: (0, b, 0)),
            pl.BlockSpec((NR, BN, D), lambda b: (0, b, 0)),
        ],
        out_shape=[
            jax.ShapeDtypeStruct((16, NP), jnp.float32),
            jax.ShapeDtypeStruct((NTY, NP, 16), jnp.float32),
            jax.ShapeDtypeStruct((NR, NP, D), jnp.float32),
        ],
    )(p0, p1, tbrw, tbrb, rele, mfrw, imw, imb, ibnw, ibnb)


def _combine_body(p0, p1, out):
    out[...] = p0[...] + p1[...]


def _combine(p0, p1):
    return pl.pallas_call(
        _combine_body,
        grid=(NP // BN,),
        in_specs=[pl.BlockSpec((BN, D), lambda b: (b, 0))] * 2,
        out_specs=pl.BlockSpec((BN, D), lambda b: (b, 0)),
        out_shape=jax.ShapeDtypeStruct((NP, D), jnp.float32),
    )(p0, p1)


def _recip_body(d_ref, out_ref):
    d = d_ref[0:8, :] + d_ref[8:16, :]
    out_ref[...] = jnp.where(d == 0.0, 0.0, 1.0 / d)


def _recip(den):
    return pl.pallas_call(
        _recip_body,
        out_shape=jax.ShapeDtypeStruct((8, NP // 8), jnp.float32),
    )(den.reshape(16, NP // 8)).reshape(NP)


_MESH = plsc.VectorSubcoreMesh(core_axis_name="c", subcore_axis_name="s")


@functools.partial(
    pl.kernel,
    mesh=_MESH,
    out_type=[
        jax.ShapeDtypeStruct((EP,), jnp.float32),
        jax.ShapeDtypeStruct((2 * NP,), jnp.float32),
        jax.ShapeDtypeStruct((EP,), jnp.int32),
    ],
    scratch_types=[
        pltpu.VMEM((EPT,), jnp.int32),
        pltpu.VMEM((NCH, C), jnp.int32),
        pltpu.VMEM((EPT,), jnp.int32),
        pltpu.VMEM((EPT,), jnp.int32),
        pltpu.VMEM((EPT,), jnp.int32),
        pltpu.VMEM((EPT,), jnp.float32),
        pltpu.VMEM((C,), jnp.int32),
        pltpu.VMEM((C,), jnp.int32),
        pltpu.VMEM((C,), jnp.int32),
        pltpu.VMEM((C,), jnp.int32),
        pltpu.VMEM((C,), jnp.float32),
        pltpu.VMEM((C,), jnp.float32),
        pltpu.VMEM((C,), jnp.float32),
        pltpu.VMEM((C,), jnp.float32),
        pltpu.VMEM((ROWS_PT,), jnp.float32),
        pltpu.VMEM_SHARED((NP,), jnp.float32),
        pltpu.SemaphoreType.DMA,
        pltpu.SemaphoreType.DMA,
    ],
)
def _sc_scores(src_h, tgt2_h, tt_h, rel_h, sflat_h, iwflat_h,
               exp_h, den_h, sidx_h, srcall, tgt2d, ttall, relall,
               sidxall, expall,
               sidx0, sidx1, iwiv0, iwiv1,
               sbuf0, sbuf1, iwbuf0, iwbuf1, zb, den_sh, sem0, sem1):
    cid = lax.axis_index("c")
    sid = lax.axis_index("s")
    wid = cid * NT_PER_CORE + sid
    base = wid * EPT
    slots = [(sidx0, iwiv0, sbuf0, iwbuf0, sem0),
             (sidx1, iwiv1, sbuf1, iwbuf1, sem1)]

    pltpu.sync_copy(src_h.at[pl.ds(base, EPT)], srcall)
    pltpu.sync_copy(tgt2_h.at[pl.ds(wid * NCH, NCH)], tgt2d)
    pltpu.sync_copy(tt_h.at[pl.ds(base, EPT)], ttall)
    pltpu.sync_copy(rel_h.at[pl.ds(base, EPT)], relall)

    def zb_init(i, _):
        zb[pl.ds(i * 16, 16)] = jnp.zeros((16,), jnp.float32)
        return 0

    lax.fori_loop(0, ROWS_PT // 16, zb_init, 0)
    pltpu.sync_copy(zb, den_sh.at[pl.ds(sid * ROWS_PT, ROWS_PT)])
    plsc.subcore_barrier()

    def issue(c, b):
        sidx, iwiv, sbuf, iwbuf, sem = slots[b]
        off = c * C
        for g in range(C // 16):
            sl = pl.ds(g * 16, 16)
            al = pl.ds(off + g * 16, 16)
            t16 = tgt2d[c, sl]
            r16 = relall[al]
            iwiv[sl] = (ttall[al] * NP + t16) * 16 + r16
            mi = r16 * NP + srcall[al]
            sidx[sl] = mi
            sidxall[al] = mi
        pltpu.async_copy(sflat_h.at[sidx], sbuf, sem)
        pltpu.async_copy(iwflat_h.at[iwiv], iwbuf, sem)

    def wait(b):
        sidx, iwiv, sbuf, iwbuf, sem = slots[b]
        pltpu.make_async_copy(sflat_h.at[sidx], sbuf, sem).wait()
        pltpu.make_async_copy(iwflat_h.at[iwiv], iwbuf, sem).wait()

    def process(c, b):
        sidx, iwiv, sbuf, iwbuf, sem = slots[b]
        off = c * C
        for g in range(C // 16):
            sl = pl.ds(g * 16, 16)
            expall[pl.ds(off + g * 16, 16)] = jnp.exp(sbuf[sl] * iwbuf[sl])
        pltpu.sync_copy(expall.at[pl.ds(off, C)],
                        den_sh.at[tgt2d.at[c]], add=True)

    issue(0, 0)

    def super_step(k, _):
        issue(2 * k + 1, 1)
        wait(0)
        process(2 * k, 0)

        @pl.when(k < NCH // 2 - 1)
        def _():
            issue(2 * k + 2, 0)

        wait(1)
        process(2 * k + 1, 1)
        return 0

    lax.fori_loop(0, NCH // 2, super_step, 0)
    pltpu.sync_copy(expall, exp_h.at[pl.ds(base, EPT)])
    pltpu.sync_copy(sidxall, sidx_h.at[pl.ds(base, EPT)])
    plsc.subcore_barrier()
    pltpu.sync_copy(den_sh.at[pl.ds(sid * ROWS_PT, ROWS_PT)],
                    den_h.at[pl.ds(cid * NP + sid * ROWS_PT, ROWS_PT)])


@functools.partial(
    pl.kernel,
    mesh=_MESH,
    out_type=[
        jax.ShapeDtypeStruct((EP,), jnp.float32),
        jax.ShapeDtypeStruct((2, NP, D), jnp.float32),
    ],
    scratch_types=[
        pltpu.VMEM((NCH, C), jnp.int32),
        pltpu.VMEM((NCH, C), jnp.int32),
        pltpu.VMEM((EPT,), jnp.float32),
        pltpu.VMEM((C,), jnp.float32),
        pltpu.VMEM((C,), jnp.float32),
        pltpu.VMEM((C, D), jnp.float32),
        pltpu.VMEM((C, D), jnp.float32),
        pltpu.VMEM_SHARED((NP, D), jnp.float32),
        pltpu.SemaphoreType.DMA,
        pltpu.SemaphoreType.DMA,
    ],
)
def _sc_messages(midx2_h, tgt2_h, exp_h, recip_h, m_h,
                 ew_h, out_h, midx2d, tgt2d, expall,
                 rb0, rb1, mrows0, mrows1, out_sh, sem0, sem1):
    cid = lax.axis_index("c")
    sid = lax.axis_index("s")
    wid = cid * NT_PER_CORE + sid
    base = wid * EPT
    slots = [(rb0, mrows0, sem0), (rb1, mrows1, sem1)]

    pltpu.sync_copy(midx2_h.at[pl.ds(wid * NCH, NCH)], midx2d)
    pltpu.sync_copy(tgt2_h.at[pl.ds(wid * NCH, NCH)], tgt2d)
    pltpu.sync_copy(exp_h.at[pl.ds(base, EPT)], expall)

    def mz(i, _):
        for j in range(D // 16):
            mrows0[i, pl.ds(j * 16, 16)] = jnp.zeros((16,), jnp.float32)
        return 0

    lax.fori_loop(0, C, mz, 0)
    for k in range(ROWS_PT // C):
        pltpu.sync_copy(mrows0, out_sh.at[pl.ds(sid * ROWS_PT + k * C, C)])
    plsc.subcore_barrier()

    def issue(c, b):
        rb, mrows, sem = slots[b]
        pltpu.async_copy(m_h.at[midx2d.at[c]], mrows, sem)
        pltpu.async_copy(recip_h.at[tgt2d.at[c]], rb, sem)

    def wait(b):
        rb, mrows, sem = slots[b]
        pltpu.make_async_copy(m_h.at[midx2d.at[0]], mrows, sem).wait()
        pltpu.make_async_copy(recip_h.at[tgt2d.at[0]], rb, sem).wait()

    def process(c, b):
        rb, mrows, sem = slots[b]
        off = c * C
        for g in range(C // 16):
            sl = pl.ds(g * 16, 16)
            al = pl.ds(off + g * 16, 16)
            expall[al] = expall[al] * rb[sl]

        def scale(g, _):
            w16 = expall[pl.ds(off + g * 16, 16)]
            for j in range(16):
                w = w16[j]
                e = g * 16 + j
                for k in range(D // 16):
                    sl = pl.ds(k * 16, 16)
                    mrows[e, sl] = mrows[e, sl] * w
            return 0

        lax.fori_loop(0, C // 16, scale, 0)
        pltpu.sync_copy(mrows, out_sh.at[tgt2d.at[c]], add=True)

    issue(0, 0)

    def super_step(k, _):
        issue(2 * k + 1, 1)
        wait(0)
        process(2 * k, 0)

        @pl.when(k < NCH // 2 - 1)
        def _():
            issue(2 * k + 2, 0)

        wait(1)
        process(2 * k + 1, 1)
        return 0

    lax.fori_loop(0, NCH // 2, super_step, 0)
    pltpu.sync_copy(expall, ew_h.at[pl.ds(base, EPT)])
    plsc.subcore_barrier()
    pltpu.sync_copy(out_sh.at[pl.ds(sid * ROWS_PT, ROWS_PT)],
                    out_h.at[cid, pl.ds(sid * ROWS_PT, ROWS_PT)])


def kernel(new_edges, new_node_types, entity_embeddings, default_emb,
           relation_emb_table, tbr_W, tbr_b, mfr_W, im_W, im_b, ibn_W, ibn_b):
    E = new_edges.shape[0] + 1
    static_edge = jnp.zeros((1, 5), jnp.int32)
    edges = jnp.concatenate([new_edges, static_edge], axis=0)
    npad = EP - E
    i = jnp.arange(npad, dtype=jnp.int32)
    src = jnp.concatenate([edges[:, 0], 10000 + (i % 16)])
    tgt = jnp.concatenate([edges[:, 2], 10016 + (i % 16)])
    tt = jnp.concatenate([edges[:, 3], jnp.zeros((npad,), jnp.int32)])
    rel = jnp.concatenate([edges[:, 4], jnp.zeros((npad,), jnp.int32)])

    emb = jnp.concatenate([default_emb[new_node_types], entity_embeddings], axis=0)
    p0 = jnp.pad(emb, ((0, NP - emb.shape[0]), (0, 0)))
    p1 = jnp.zeros((NP, D), jnp.float32)

    tgt2d = tgt.reshape(EP // C, C)
    ews = []
    for _ in range(3):
        srows, iwp, m = _dense(p0, p1, tbr_W, tbr_b, relation_emb_table,
                               mfr_W, im_W, im_b, ibn_W, ibn_b)
        sflat = srows.reshape(16 * NP)
        iwflat = iwp.reshape(NTY * NP * 16)
        mflat = m.reshape(NR * NP, D)
        exp_s, denoms, midx = _sc_scores(src, tgt2d, tt, rel, sflat, iwflat)
        recip = _recip(denoms)
        ew, outp = _sc_messages(midx.reshape(EP // C, C), tgt2d, exp_s,
                                recip, mflat)
        p0, p1 = outp[0], outp[1]
        ews.append(ew)

    ent = _combine(p0, p1)
    return ent[:10000], jnp.stack(ews)[:, :E]


# speedup vs baseline: 1.0577x; 1.0073x over previous
import functools

import jax
import jax.numpy as jnp
from jax import lax
from jax.experimental import pallas as pl
from jax.experimental.pallas import tpu as pltpu
from jax.experimental.pallas import tpu_sc as plsc

NP = 10240
EP = 163840
NWORK = 32
EPT = EP // NWORK
C = 128
NCH = EPT // C
NT_PER_CORE = 16
ROWS_PT = NP // NT_PER_CORE
NR = 9
NTY = 3
D = 128
RD = 64
NI = 8
BN = 2048


def _selu(x):
    alpha = 1.6732632423543772848170429916717
    scale = 1.0507009873554804934193349852946
    return scale * jnp.where(x > 0, x, alpha * (jnp.exp(x) - 1.0))


def _softmax(x):
    m = jnp.max(x, axis=-1, keepdims=True)
    e = jnp.exp(x - m)
    return e / jnp.sum(e, axis=-1, keepdims=True)


def _full_spec(shape):
    return pl.BlockSpec(shape, lambda b: tuple(0 for _ in shape))


def _dense_body(p0, p1, tbrw, tbrb, rele, mfrw, imw, imb, ibnw, ibnb,
                s_out, iw_out, m_out):
    emb = p0[...] + p1[...]
    row16 = lax.broadcasted_iota(jnp.int32, (16, BN), 0)
    s_acc = jnp.zeros((16, BN), jnp.float32)
    m_rows = []
    for r in range(NR):
        h = _selu(jnp.dot(emb, tbrw[r], preferred_element_type=jnp.float32)
                  + tbrb[r][None, :])
        s_r = jnp.sum(h * rele[r][None, :], axis=-1)
        s_acc = s_acc + jnp.where(row16 == r, s_r[None, :], 0.0)
        m_rows.append(jnp.dot(emb, mfrw[r], preferred_element_type=jnp.float32))
    s_out[...] = s_acc
    m_out[...] = jnp.stack(m_rows, axis=0)
    iw_rows = []
    for t in range(NTY):
        intents = _softmax(jnp.dot(emb, imw[t], preferred_element_type=jnp.float32)
                           + imb[t][None, :])
        w = _softmax(jnp.dot(intents, ibnw[t], preferred_element_type=jnp.float32)
                     + ibnb[t][None, :])
        iw_rows.append(jnp.concatenate(
            [w, jnp.zeros((BN, 16 - NR), jnp.float32)], axis=1))
    iw_out[...] = jnp.stack(iw_rows, axis=0)


def _dense(p0, p1, tbrw, tbrb, rele, mfrw, imw, imb, ibnw, ibnb):
    return pl.pallas_call(
        _dense_body,
        grid=(NP // BN,),
        in_specs=[
            pl.BlockSpec((BN, D), lambda b: (b, 0)),
            pl.BlockSpec((BN, D), lambda b: (b, 0)),
            _full_spec((NR, D, RD)), _full_spec((NR, RD)), _full_spec((NR, RD)),
            _full_spec((NR, D, D)), _full_spec((NTY, D, NI)),
            _full_spec((NTY, NI)), _full_spec((NTY, NI, NR)),
            _full_spec((NTY, NR)),
        ],
        out_specs=[
            pl.BlockSpec((16, BN), lambda b: (0, b)),
            pl.BlockSpec((NTY, BN, 16), lambda b: (0, b, 0)),
            pl.BlockSpec((NR, BN, D), lambda b: (0, b, 0)),
        ],
        out_shape=[
            jax.ShapeDtypeStruct((16, NP), jnp.float32),
            jax.ShapeDtypeStruct((NTY, NP, 16), jnp.float32),
            jax.ShapeDtypeStruct((NR, NP, D), jnp.float32),
        ],
    )(p0, p1, tbrw, tbrb, rele, mfrw, imw, imb, ibnw, ibnb)


def _combine_body(p0, p1, out):
    out[...] = p0[...] + p1[...]


def _combine(p0, p1):
    return pl.pallas_call(
        _combine_body,
        grid=(NP // BN,),
        in_specs=[pl.BlockSpec((BN, D), lambda b: (b, 0))] * 2,
        out_specs=pl.BlockSpec((BN, D), lambda b: (b, 0)),
        out_shape=jax.ShapeDtypeStruct((NP, D), jnp.float32),
    )(p0, p1)


def _recip_body(d_ref, out_ref):
    d = d_ref[0:8, :] + d_ref[8:16, :]
    out_ref[...] = jnp.where(d == 0.0, 0.0, 1.0 / d)


def _recip(den):
    return pl.pallas_call(
        _recip_body,
        out_shape=jax.ShapeDtypeStruct((8, NP // 8), jnp.float32),
    )(den.reshape(16, NP // 8)).reshape(NP)


_MESH = plsc.VectorSubcoreMesh(core_axis_name="c", subcore_axis_name="s")


@functools.partial(
    pl.kernel,
    mesh=_MESH,
    out_type=[
        jax.ShapeDtypeStruct((EP,), jnp.float32),
        jax.ShapeDtypeStruct((2 * NP,), jnp.float32),
        jax.ShapeDtypeStruct((EP,), jnp.int32),
    ],
    scratch_types=[
        pltpu.VMEM((EPT,), jnp.int32),
        pltpu.VMEM((NCH, C), jnp.int32),
        pltpu.VMEM((EPT,), jnp.int32),
        pltpu.VMEM((EPT,), jnp.int32),
        pltpu.VMEM((EPT,), jnp.int32),
        pltpu.VMEM((EPT,), jnp.float32),
        pltpu.VMEM((C,), jnp.int32),
        pltpu.VMEM((C,), jnp.int32),
        pltpu.VMEM((C,), jnp.int32),
        pltpu.VMEM((C,), jnp.int32),
        pltpu.VMEM((C,), jnp.float32),
        pltpu.VMEM((C,), jnp.float32),
        pltpu.VMEM((C,), jnp.float32),
        pltpu.VMEM((C,), jnp.float32),
        pltpu.VMEM((ROWS_PT,), jnp.float32),
        pltpu.VMEM_SHARED((NP,), jnp.float32),
        pltpu.SemaphoreType.DMA,
        pltpu.SemaphoreType.DMA,
    ],
)
def _sc_scores(src_h, tgt2_h, tt_h, rel_h, sflat_h, iwflat_h,
               exp_h, den_h, sidx_h, srcall, tgt2d, ttall, relall,
               sidxall, expall,
               sidx0, sidx1, iwiv0, iwiv1,
               sbuf0, sbuf1, iwbuf0, iwbuf1, zb, den_sh, sem0, sem1):
    cid = lax.axis_index("c")
    sid = lax.axis_index("s")
    wid = cid * NT_PER_CORE + sid
    base = wid * EPT
    slots = [(sidx0, iwiv0, sbuf0, iwbuf0, sem0),
             (sidx1, iwiv1, sbuf1, iwbuf1, sem1)]

    pltpu.sync_copy(src_h.at[pl.ds(base, EPT)], srcall)
    pltpu.sync_copy(tgt2_h.at[pl.ds(wid * NCH, NCH)], tgt2d)
    pltpu.sync_copy(tt_h.at[pl.ds(base, EPT)], ttall)
    pltpu.sync_copy(rel_h.at[pl.ds(base, EPT)], relall)

    def zb_init(i, _):
        zb[pl.ds(i * 16, 16)] = jnp.zeros((16,), jnp.float32)
        return 0

    lax.fori_loop(0, ROWS_PT // 16, zb_init, 0)
    pltpu.sync_copy(zb, den_sh.at[pl.ds(sid * ROWS_PT, ROWS_PT)])
    plsc.subcore_barrier()

    def issue(c, b):
        sidx, iwiv, sbuf, iwbuf, sem = slots[b]
        off = c * C
        for g in range(C // 16):
            sl = pl.ds(g * 16, 16)
            al = pl.ds(off + g * 16, 16)
            t16 = tgt2d[c, sl]
            r16 = relall[al]
            iwiv[sl] = (ttall[al] * NP + t16) * 16 + r16
            mi = r16 * NP + srcall[al]
            sidx[sl] = mi
            sidxall[al] = mi
        pltpu.async_copy(sflat_h.at[sidx], sbuf, sem)
        pltpu.async_copy(iwflat_h.at[iwiv], iwbuf, sem)

    def wait(b):
        sidx, iwiv, sbuf, iwbuf, sem = slots[b]
        pltpu.make_async_copy(sflat_h.at[sidx], sbuf, sem).wait()
        pltpu.make_async_copy(iwflat_h.at[iwiv], iwbuf, sem).wait()

    def process(c, b):
        sidx, iwiv, sbuf, iwbuf, sem = slots[b]
        off = c * C
        for g in range(C // 16):
            sl = pl.ds(g * 16, 16)
            expall[pl.ds(off + g * 16, 16)] = jnp.exp(sbuf[sl] * iwbuf[sl])
        pltpu.sync_copy(expall.at[pl.ds(off, C)],
                        den_sh.at[tgt2d.at[c]], add=True)

    issue(0, 0)

    def super_step(k, _):
        issue(2 * k + 1, 1)
        wait(0)
        process(2 * k, 0)

        @pl.when(k < NCH // 2 - 1)
        def _():
            issue(2 * k + 2, 0)

        wait(1)
        process(2 * k + 1, 1)
        return 0

    lax.fori_loop(0, NCH // 2, super_step, 0)
    pltpu.sync_copy(expall, exp_h.at[pl.ds(base, EPT)])
    pltpu.sync_copy(sidxall, sidx_h.at[pl.ds(base, EPT)])
    plsc.subcore_barrier()
    pltpu.sync_copy(den_sh.at[pl.ds(sid * ROWS_PT, ROWS_PT)],
                    den_h.at[pl.ds(cid * NP + sid * ROWS_PT, ROWS_PT)])


@functools.partial(
    pl.kernel,
    mesh=_MESH,
    out_type=[
        jax.ShapeDtypeStruct((EP,), jnp.float32),
        jax.ShapeDtypeStruct((2, NP, D), jnp.float32),
    ],
    scratch_types=[
        pltpu.VMEM((NCH, C), jnp.int32),
        pltpu.VMEM((NCH, C), jnp.int32),
        pltpu.VMEM((EPT,), jnp.float32),
        pltpu.VMEM((C,), jnp.float32),
        pltpu.VMEM((C,), jnp.float32),
        pltpu.VMEM((C, D), jnp.float32),
        pltpu.VMEM((C, D), jnp.float32),
        pltpu.VMEM_SHARED((NP, D), jnp.float32),
        pltpu.SemaphoreType.DMA,
        pltpu.SemaphoreType.DMA,
    ],
)
def _sc_messages(midx2_h, tgt2_h, exp_h, recip_h, m_h,
                 ew_h, out_h, midx2d, tgt2d, expall,
                 rb0, rb1, mrows0, mrows1, out_sh, sem0, sem1):
    cid = lax.axis_index("c")
    sid = lax.axis_index("s")
    wid = cid * NT_PER_CORE + sid
    base = wid * EPT
    slots = [(rb0, mrows0, sem0), (rb1, mrows1, sem1)]

    pltpu.sync_copy(midx2_h.at[pl.ds(wid * NCH, NCH)], midx2d)
    pltpu.sync_copy(tgt2_h.at[pl.ds(wid * NCH, NCH)], tgt2d)
    pltpu.sync_copy(exp_h.at[pl.ds(base, EPT)], expall)

    def mz(i, _):
        for j in range(D // 16):
            mrows0[i, pl.ds(j * 16, 16)] = jnp.zeros((16,), jnp.float32)
        return 0

    lax.fori_loop(0, C, mz, 0)
    for k in range(ROWS_PT // C):
        pltpu.sync_copy(mrows0, out_sh.at[pl.ds(sid * ROWS_PT + k * C, C)])
    plsc.subcore_barrier()

    def issue(c, b):
        rb, mrows, sem = slots[b]
        pltpu.async_copy(m_h.at[midx2d.at[c]], mrows, sem)
        pltpu.async_copy(recip_h.at[tgt2d.at[c]], rb, sem)

    def wait(b):
        rb, mrows, sem = slots[b]
        pltpu.make_async_copy(m_h.at[midx2d.at[0]], mrows, sem).wait()
        pltpu.make_async_copy(recip_h.at[tgt2d.at[0]], rb, sem).wait()

    def process(c, b):
        rb, mrows, sem = slots[b]
        off = c * C
        for g in range(C // 16):
            sl = pl.ds(g * 16, 16)
            al = pl.ds(off + g * 16, 16)
            expall[al] = expall[al] * rb[sl]

        def scale(g, _):
            w16 = expall[pl.ds(off + g * 16, 16)]
            for j in range(16):
                w = w16[j]
                e = g * 16 + j
                for k in range(D // 16):
                    sl = pl.ds(k * 16, 16)
                    mrows[e, sl] = mrows[e, sl] * w
            return 0

        lax.fori_loop(0, C // 16, scale, 0)
        pltpu.sync_copy(mrows, out_sh.at[tgt2d.at[c]], add=True)

    issue(0, 0)

    def super_step(k, _):
        issue(2 * k + 1, 1)
        wait(0)
        process(2 * k, 0)

        @pl.when(k < NCH // 2 - 1)
        def _():
            issue(2 * k + 2, 0)

        wait(1)
        process(2 * k + 1, 1)
        return 0

    lax.fori_loop(0, NCH // 2, super_step, 0)
    pltpu.sync_copy(expall, ew_h.at[pl.ds(base, EPT)])
    plsc.subcore_barrier()
    pltpu.sync_copy(out_sh.at[pl.ds(sid * ROWS_PT, ROWS_PT)],
                    out_h.at[cid, pl.ds(sid * ROWS_PT, ROWS_PT)])


def kernel(new_edges, new_node_types, entity_embeddings, default_emb,
           relation_emb_table, tbr_W, tbr_b, mfr_W, im_W, im_b, ibn_W, ibn_b):
    E = new_edges.shape[0] + 1
    static_edge = jnp.zeros((1, 5), jnp.int32)
    edges = jnp.concatenate([new_edges, static_edge], axis=0)
    npad = EP - E
    i = jnp.arange(npad, dtype=jnp.int32)
    src = jnp.concatenate([edges[:, 0], 10000 + (i % 16)])
    tgt = jnp.concatenate([edges[:, 2], 10016 + (i % 16)])
    tt = jnp.concatenate([edges[:, 3], jnp.zeros((npad,), jnp.int32)])
    rel = jnp.concatenate([edges[:, 4], jnp.zeros((npad,), jnp.int32)])

    emb = jnp.concatenate([default_emb[new_node_types], entity_embeddings], axis=0)
    p0 = jnp.pad(emb, ((0, NP - emb.shape[0]), (0, 0)))
    p1 = jnp.zeros((NP, D), jnp.float32)

    tgt2d = tgt.reshape(EP // C, C)
    ews = []
    for _ in range(3):
        srows, iwp, m = _dense(p0, p1, tbr_W, tbr_b, relation_emb_table,
                               mfr_W, im_W, im_b, ibn_W, ibn_b)
        sflat = srows.reshape(16 * NP)
        iwflat = iwp.reshape(NTY * NP * 16)
        mflat = m.reshape(NR * NP, D)
        exp_s, denoms, midx = _sc_scores(src, tgt2d, tt, rel, sflat, iwflat)
        recip = _recip(denoms)
        ew, outp = _sc_messages(midx.reshape(EP // C, C), tgt2d, exp_s,
                                recip, mflat)
        p0, p1 = outp[0], outp[1]
        ews.append(ew)

    ent = _combine(p0, p1)
    return ent[:10000], jnp.stack(ews)[:, :E]
